# Initial kernel scaffold; baseline (speedup 1.0000x reference)
import jax
import jax.numpy as jnp
from jax.experimental import pallas as pl


def kernel(gt_vertices, est_vertices, gt_connections, est_connections):
    raise NotImplementedError("write your pallas kernel here")


# speedup vs baseline: 5.6953x; 5.6953x over previous
import functools

import jax
import jax.numpy as jnp
from jax import lax
from jax.experimental import pallas as pl
from jax.experimental.pallas import tpu as pltpu
from jax.experimental.pallas import tpu_sc as plsc

_CH = 1024
_L = 16


def _build(B, P, E_pad, nchunk):
    info = plsc.get_sparse_core_info()
    NC, NS = info.num_cores, info.num_subcores
    NW = NC * NS
    assert NW == B, (NW, B)

    mesh = plsc.VectorSubcoreMesh(core_axis_name="c", subcore_axis_name="s")

    @functools.partial(
        pl.kernel,
        mesh=mesh,
        out_type=jax.ShapeDtypeStruct((NW, _L), jnp.float32),
        compiler_params=pltpu.CompilerParams(needs_layout_passes=False),
        scratch_types=[
            pltpu.VMEM((P,), jnp.float32),
            pltpu.VMEM((P,), jnp.float32),
            pltpu.VMEM((4, _CH), jnp.int32),
            pltpu.VMEM((_L,), jnp.float32),
        ],
    )
    def edge_loss(gt_hbm, est_hbm, conn_hbm, out_hbm, gt_v, est_v, conn_v, acc_v):
        wid = lax.axis_index("s") * NC + lax.axis_index("c")
        pltpu.sync_copy(gt_hbm.at[wid], gt_v)
        pltpu.sync_copy(est_hbm.at[wid], est_v)

        def chunk_body(k, acc):
            pltpu.sync_copy(conn_hbm.at[k], conn_v)

            def group(g, acc):
                base = g * _L
                a0 = conn_v[0, pl.ds(base, _L)] * 3
                a1 = conn_v[1, pl.ds(base, _L)] * 3
                b0 = conn_v[2, pl.ds(base, _L)] * 3
                b1 = conn_v[3, pl.ds(base, _L)] * 3
                for d in range(3):
                    g1 = plsc.load_gather(gt_v, [a1 + d])
                    g0 = plsc.load_gather(gt_v, [a0 + d])
                    e1 = plsc.load_gather(est_v, [b1 + d])
                    e0 = plsc.load_gather(est_v, [b0 + d])
                    s = (g1 - g0) - (e1 - e0)
                    acc = acc + s * s
                return acc

            return lax.fori_loop(0, _CH // _L, group, acc)

        acc = lax.fori_loop(0, nchunk, chunk_body, jnp.zeros((_L,), jnp.float32))
        acc_v[...] = acc
        pltpu.sync_copy(acc_v, out_hbm.at[wid])

    return edge_loss


def kernel(gt_vertices, est_vertices, gt_connections, est_connections):
    B, N, _ = gt_vertices.shape
    E = gt_connections.shape[0]
    nchunk = -(-E // _CH)
    E_pad = nchunk * _CH

    conn = jnp.stack(
        [gt_connections[:, 0], gt_connections[:, 1],
         est_connections[:, 0], est_connections[:, 1]], axis=0)
    conn = jnp.pad(conn, ((0, 0), (0, E_pad - E)))
    conn = conn.reshape(4, nchunk, _CH).transpose(1, 0, 2)

    P = -(-(N * 3) // 16) * 16
    gt_flat = jnp.pad(gt_vertices.reshape(B, N * 3), ((0, 0), (0, P - N * 3)))
    est_flat = jnp.pad(est_vertices.reshape(B, N * 3), ((0, 0), (0, P - N * 3)))

    fn = _build(B, P, E_pad, nchunk)
    partials = fn(gt_flat, est_flat, conn)
    return jnp.sum(partials)


# speedup vs baseline: 6.3142x; 1.1087x over previous
import functools

import jax
import jax.numpy as jnp
from jax import lax
from jax.experimental import pallas as pl
from jax.experimental.pallas import tpu as pltpu
from jax.experimental.pallas import tpu_sc as plsc

_CH = 1024
_L = 16


def _build(B, P, E_pad, nchunk):
    info = plsc.get_sparse_core_info()
    NC, NS = info.num_cores, info.num_subcores
    NW = NC * NS
    assert NW == B, (NW, B)

    mesh = plsc.VectorSubcoreMesh(core_axis_name="c", subcore_axis_name="s")

    @functools.partial(
        pl.kernel,
        mesh=mesh,
        out_type=jax.ShapeDtypeStruct((NW, _L), jnp.float32),
        compiler_params=pltpu.CompilerParams(needs_layout_passes=False),
        scratch_types=[
            pltpu.VMEM((P,), jnp.float32),
            pltpu.VMEM((P,), jnp.float32),
            pltpu.VMEM((2, 4, _CH), jnp.int32),
            pltpu.VMEM((_L,), jnp.float32),
            pltpu.SemaphoreType.DMA,
            pltpu.SemaphoreType.DMA,
            pltpu.SemaphoreType.DMA((2,)),
        ],
    )
    def edge_loss(gt_hbm, est_hbm, conn_hbm, out_hbm,
                  gt_v, est_v, conn_v, acc_v, gt_sem, est_sem, conn_sems):
        wid = lax.axis_index("s") * NC + lax.axis_index("c")
        gt_cp = pltpu.make_async_copy(gt_hbm.at[wid], gt_v, gt_sem)
        est_cp = pltpu.make_async_copy(est_hbm.at[wid], est_v, est_sem)
        gt_cp.start()
        est_cp.start()
        pltpu.make_async_copy(conn_hbm.at[0], conn_v.at[0],
                              conn_sems.at[0]).start()
        gt_cp.wait()
        est_cp.wait()

        def compute_group(buf, base, acc):
            a0 = conn_v[buf, 0, pl.ds(base, _L)] * 3
            a1 = conn_v[buf, 1, pl.ds(base, _L)] * 3
            b0 = conn_v[buf, 2, pl.ds(base, _L)] * 3
            b1 = conn_v[buf, 3, pl.ds(base, _L)] * 3
            for d in range(3):
                g1 = plsc.load_gather(gt_v, [a1 + d])
                g0 = plsc.load_gather(gt_v, [a0 + d])
                e1 = plsc.load_gather(est_v, [b1 + d])
                e0 = plsc.load_gather(est_v, [b0 + d])
                s = (g1 - g0) - (e1 - e0)
                acc = acc + s * s
            return acc

        def chunk_body(k, acc):
            buf = lax.rem(k, 2)
            nxt = 1 - buf
            pltpu.make_async_copy(conn_hbm.at[k], conn_v.at[buf],
                                  conn_sems.at[buf]).wait()

            @pl.when(k + 1 < nchunk)
            def _():
                pltpu.make_async_copy(conn_hbm.at[k + 1], conn_v.at[nxt],
                                      conn_sems.at[nxt]).start()

            def group(g, acc):
                base = g * (4 * _L)
                for u in range(4):
                    acc = compute_group(buf, base + u * _L, acc)
                return acc

            return lax.fori_loop(0, _CH // (4 * _L), group, acc)

        acc = lax.fori_loop(0, nchunk, chunk_body, jnp.zeros((_L,), jnp.float32))
        acc_v[...] = acc
        pltpu.sync_copy(acc_v, out_hbm.at[wid])

    return edge_loss


def kernel(gt_vertices, est_vertices, gt_connections, est_connections):
    B, N, _ = gt_vertices.shape
    E = gt_connections.shape[0]
    nchunk = -(-E // _CH)
    E_pad = nchunk * _CH

    conn = jnp.stack(
        [gt_connections[:, 0], gt_connections[:, 1],
         est_connections[:, 0], est_connections[:, 1]], axis=0)
    conn = jnp.pad(conn, ((0, 0), (0, E_pad - E)))
    conn = conn.reshape(4, nchunk, _CH).transpose(1, 0, 2)

    P = -(-(N * 3) // 16) * 16
    gt_flat = jnp.pad(gt_vertices.reshape(B, N * 3), ((0, 0), (0, P - N * 3)))
    est_flat = jnp.pad(est_vertices.reshape(B, N * 3), ((0, 0), (0, P - N * 3)))

    fn = _build(B, P, E_pad, nchunk)
    partials = fn(gt_flat, est_flat, conn)
    return jnp.sum(partials)


# speedup vs baseline: 6.4622x; 1.0234x over previous
import functools

import jax
import jax.numpy as jnp
from jax import lax
from jax.experimental import pallas as pl
from jax.experimental.pallas import tpu as pltpu
from jax.experimental.pallas import tpu_sc as plsc

_CH = 1024
_L = 16


def _build(B, P, E_pad, nchunk):
    info = plsc.get_sparse_core_info()
    NC, NS = info.num_cores, info.num_subcores
    NW = NC * NS
    assert NW == B, (NW, B)

    mesh = plsc.VectorSubcoreMesh(core_axis_name="c", subcore_axis_name="s")

    @functools.partial(
        pl.kernel,
        mesh=mesh,
        out_type=jax.ShapeDtypeStruct((NW, _L), jnp.float32),
        compiler_params=pltpu.CompilerParams(needs_layout_passes=False),
        scratch_types=[
            pltpu.VMEM((P,), jnp.float32),
            pltpu.VMEM((P,), jnp.float32),
            pltpu.VMEM((2, 4, _CH), jnp.int32),
            pltpu.VMEM((_L,), jnp.float32),
            pltpu.SemaphoreType.DMA,
            pltpu.SemaphoreType.DMA,
            pltpu.SemaphoreType.DMA((2,)),
        ],
    )
    def edge_loss(gt_hbm, est_hbm, conn_hbm, out_hbm,
                  gt_v, est_v, conn_v, acc_v, gt_sem, est_sem, conn_sems):
        wid = lax.axis_index("s") * NC + lax.axis_index("c")
        gt_cp = pltpu.make_async_copy(gt_hbm.at[wid], gt_v, gt_sem)
        est_cp = pltpu.make_async_copy(est_hbm.at[wid], est_v, est_sem)
        gt_cp.start()
        est_cp.start()
        pltpu.make_async_copy(conn_hbm.at[0], conn_v.at[0],
                              conn_sems.at[0]).start()
        gt_cp.wait()
        est_cp.wait()

        def compute_group(buf, base, acc):
            a0 = conn_v[buf, 0, pl.ds(base, _L)] * 3
            a1 = conn_v[buf, 1, pl.ds(base, _L)] * 3
            b0 = conn_v[buf, 2, pl.ds(base, _L)] * 3
            b1 = conn_v[buf, 3, pl.ds(base, _L)] * 3
            for d in range(3):
                g1 = plsc.load_gather(gt_v, [a1 + d])
                g0 = plsc.load_gather(gt_v, [a0 + d])
                e1 = plsc.load_gather(est_v, [b1 + d])
                e0 = plsc.load_gather(est_v, [b0 + d])
                s = (g1 - g0) - (e1 - e0)
                acc = acc + s * s
            return acc

        def chunk_body(k, acc):
            buf = lax.rem(k, 2)
            nxt = 1 - buf
            pltpu.make_async_copy(conn_hbm.at[k], conn_v.at[buf],
                                  conn_sems.at[buf]).wait()

            @pl.when(k + 1 < nchunk)
            def _():
                pltpu.make_async_copy(conn_hbm.at[k + 1], conn_v.at[nxt],
                                      conn_sems.at[nxt]).start()

            def group(g, acc):
                base = g * (4 * _L)
                for u in range(4):
                    acc = compute_group(buf, base + u * _L, acc)
                return acc

            return lax.fori_loop(0, _CH // (4 * _L), group, acc)

        acc = lax.fori_loop(0, nchunk, chunk_body, jnp.zeros((_L,), jnp.float32))
        acc_v[...] = acc
        pltpu.sync_copy(acc_v, out_hbm.at[wid])

    return edge_loss


def kernel(gt_vertices, est_vertices, gt_connections, est_connections):
    B, N, _ = gt_vertices.shape
    E = gt_connections.shape[0]
    nchunk = -(-E // _CH)
    E_pad = nchunk * _CH

    conn = jnp.stack(
        [gt_connections[:, 0], gt_connections[:, 1],
         est_connections[:, 0], est_connections[:, 1]], axis=0)
    conn = jnp.pad(conn, ((0, 0), (0, E_pad - E)))
    conn = conn.reshape(4, nchunk, _CH).transpose(1, 0, 2)

    fn = _build(B, N * 3, E_pad, nchunk)
    partials = fn(gt_vertices.reshape(B, N * 3), est_vertices.reshape(B, N * 3),
                  conn)
    return jnp.sum(partials)


# speedup vs baseline: 13.9781x; 2.1631x over previous
import functools

import jax
import jax.numpy as jnp
from jax import lax
from jax.experimental import pallas as pl
from jax.experimental.pallas import tpu as pltpu
from jax.experimental.pallas import tpu_sc as plsc

_CH = 1024
_L = 16


def _build(B, P, E_pad, nchunk):
    info = plsc.get_sparse_core_info()
    NC, NS = info.num_cores, info.num_subcores
    NW = NC * NS
    assert NW == B, (NW, B)

    mesh = plsc.VectorSubcoreMesh(core_axis_name="c", subcore_axis_name="s")

    @functools.partial(
        pl.kernel,
        mesh=mesh,
        out_type=jax.ShapeDtypeStruct((NW, _L), jnp.float32),
        compiler_params=pltpu.CompilerParams(needs_layout_passes=False,
                                             use_tc_tiling_on_sc=False),
        scratch_types=[
            pltpu.VMEM((3, P), jnp.float32),
            pltpu.VMEM((3, P), jnp.float32),
            pltpu.VMEM((2, 4, _CH), jnp.int32),
            pltpu.VMEM((_L,), jnp.float32),
            pltpu.SemaphoreType.DMA,
            pltpu.SemaphoreType.DMA,
            pltpu.SemaphoreType.DMA((2,)),
        ],
    )
    def edge_loss(gt_hbm, est_hbm, conn_hbm, out_hbm,
                  gt_v, est_v, conn_v, acc_v, gt_sem, est_sem, conn_sems):
        wid = lax.axis_index("s") * NC + lax.axis_index("c")
        vert_cps = []
        for d in range(3):
            vert_cps.append(pltpu.make_async_copy(
                gt_hbm.at[d, wid], gt_v.at[d], gt_sem))
            vert_cps.append(pltpu.make_async_copy(
                est_hbm.at[d, wid], est_v.at[d], est_sem))
        for cp in vert_cps:
            cp.start()
        pltpu.make_async_copy(conn_hbm.at[0], conn_v.at[0],
                              conn_sems.at[0]).start()
        for cp in vert_cps:
            cp.wait()

        def compute_group(buf, base, acc):
            a0 = conn_v[buf, 0, pl.ds(base, _L)]
            a1 = conn_v[buf, 1, pl.ds(base, _L)]
            b0 = conn_v[buf, 2, pl.ds(base, _L)]
            b1 = conn_v[buf, 3, pl.ds(base, _L)]
            for d in range(3):
                dv = jnp.full((_L,), d, jnp.int32)
                g1 = plsc.load_gather(gt_v, [dv, a1])
                g0 = plsc.load_gather(gt_v, [dv, a0])
                e1 = plsc.load_gather(est_v, [dv, b1])
                e0 = plsc.load_gather(est_v, [dv, b0])
                s = (g1 - g0) - (e1 - e0)
                acc = acc + s * s
            return acc

        def chunk_body(k, acc):
            buf = lax.rem(k, 2)
            nxt = 1 - buf
            pltpu.make_async_copy(conn_hbm.at[k], conn_v.at[buf],
                                  conn_sems.at[buf]).wait()

            @pl.when(k + 1 < nchunk)
            def _():
                pltpu.make_async_copy(conn_hbm.at[k + 1], conn_v.at[nxt],
                                      conn_sems.at[nxt]).start()

            def group(g, acc):
                base = g * (4 * _L)
                for u in range(4):
                    acc = compute_group(buf, base + u * _L, acc)
                return acc

            return lax.fori_loop(0, _CH // (4 * _L), group, acc)

        acc = lax.fori_loop(0, nchunk, chunk_body, jnp.zeros((_L,), jnp.float32))
        acc_v[...] = acc
        pltpu.sync_copy(acc_v, out_hbm.at[wid])

    return edge_loss


def kernel(gt_vertices, est_vertices, gt_connections, est_connections):
    B, N, _ = gt_vertices.shape
    E = gt_connections.shape[0]
    nchunk = -(-E // _CH)
    E_pad = nchunk * _CH

    conn = jnp.stack(
        [gt_connections[:, 0], gt_connections[:, 1],
         est_connections[:, 0], est_connections[:, 1]], axis=0)
    conn = jnp.pad(conn, ((0, 0), (0, E_pad - E)))
    conn = conn.reshape(4, nchunk, _CH).transpose(1, 0, 2)

    fn = _build(B, N, E_pad, nchunk)
    partials = fn(jnp.transpose(gt_vertices, (2, 0, 1)),
                  jnp.transpose(est_vertices, (2, 0, 1)), conn)
    return jnp.sum(partials)
